# baseline (device time: 103917 ns/iter reference)
import jax
import jax.numpy as jnp
from jax import lax
from jax.experimental import pallas as pl
from jax.experimental.pallas import tpu as pltpu

N_DEV = 16
SQ = 256
D_MODEL = 1024
SKV = 4096
H_PER = 8
DH = 128
ROWS = SQ // N_DEV
SCALE = 0.08838834764831843


def _bf(x):
    return x.astype(jnp.bfloat16)


def _fused_body(x_ref, wq_ref, wo_ref, k_hbm, v_hbm, out_ref,
                k_buf, v_buf, part_ref, p1_buf,
                copy_sems, p1_sems, p2_sems, send_sems):
    me = lax.axis_index("i")
    head0 = me * H_PER

    kcp = pltpu.make_async_copy(
        k_hbm.at[0, :, pl.ds(head0, H_PER), :], k_buf, copy_sems.at[0])
    vcp = pltpu.make_async_copy(
        v_hbm.at[0, :, pl.ds(head0, H_PER), :], v_buf, copy_sems.at[1])
    kcp.start()
    vcp.start()

    q = jnp.dot(_bf(x_ref[0]), _bf(wq_ref[...]),
                preferred_element_type=jnp.float32)

    kcp.wait()
    vcp.wait()

    partial = jnp.zeros((SQ, D_MODEL), jnp.float32)
    for h in range(H_PER):
        qh = _bf(q[:, h * DH:(h + 1) * DH])
        kh = _bf(k_buf[:, h, :])
        vh = _bf(v_buf[:, h, :])
        s = lax.dot_general(
            qh, kh, (((1,), (1,)), ((), ())),
            preferred_element_type=jnp.float32) * SCALE
        m = jnp.max(s, axis=1, keepdims=True)
        p = jnp.exp(s - m)
        l = jnp.sum(p, axis=1, keepdims=True)
        oh = jnp.dot(_bf(p), vh, preferred_element_type=jnp.float32) / l
        partial = partial + jnp.dot(
            _bf(oh), _bf(wo_ref[pl.ds(h * DH, DH), :]),
            preferred_element_type=jnp.float32)
    part_ref[...] = partial

    p1_descs = []
    for k in range(1, N_DEV):
        peer = lax.rem(me + k, N_DEV)
        d = pltpu.make_async_remote_copy(
            src_ref=part_ref.at[pl.ds(peer * ROWS, ROWS), :],
            dst_ref=p1_buf.at[N_DEV - k],
            send_sem=send_sems.at[0, k],
            recv_sem=p1_sems.at[N_DEV - k],
            device_id=(peer,),
            device_id_type=pl.DeviceIdType.MESH,
        )
        d.start()
        p1_descs.append(d)

    acc = part_ref[pl.ds(me * ROWS, ROWS), :]
    for k in range(1, N_DEV):
        r = pltpu.make_async_remote_copy(
            src_ref=part_ref.at[pl.ds(0, ROWS), :],
            dst_ref=p1_buf.at[k],
            send_sem=send_sems.at[0, 0],
            recv_sem=p1_sems.at[k],
            device_id=(me,),
            device_id_type=pl.DeviceIdType.MESH,
        )
        r.wait_recv()
        acc = acc + p1_buf[k]
    out_ref[0, pl.ds(me * ROWS, ROWS), :] = acc

    p2_descs = []
    for k in range(1, N_DEV):
        peer = lax.rem(me + k, N_DEV)
        d = pltpu.make_async_remote_copy(
            src_ref=out_ref.at[0, pl.ds(me * ROWS, ROWS), :],
            dst_ref=out_ref.at[0, pl.ds(me * ROWS, ROWS), :],
            send_sem=send_sems.at[1, k],
            recv_sem=p2_sems.at[N_DEV - k],
            device_id=(peer,),
            device_id_type=pl.DeviceIdType.MESH,
        )
        d.start()
        p2_descs.append(d)

    for k in range(1, N_DEV):
        src_chunk = lax.rem(me + k, N_DEV)
        r = pltpu.make_async_remote_copy(
            src_ref=out_ref.at[0, pl.ds(0, ROWS), :],
            dst_ref=out_ref.at[0, pl.ds(src_chunk * ROWS, ROWS), :],
            send_sem=send_sems.at[1, 0],
            recv_sem=p2_sems.at[k],
            device_id=(me,),
            device_id_type=pl.DeviceIdType.MESH,
        )
        r.wait_recv()

    for d in p1_descs + p2_descs:
        d.wait_send()


def kernel(x, Wq, Wo, K_ext, V_ext):
    return pl.pallas_call(
        _fused_body,
        out_shape=jax.ShapeDtypeStruct((1, SQ, D_MODEL), jnp.float32),
        in_specs=[
            pl.BlockSpec(memory_space=pltpu.VMEM),
            pl.BlockSpec(memory_space=pltpu.VMEM),
            pl.BlockSpec(memory_space=pltpu.VMEM),
            pl.BlockSpec(memory_space=pltpu.MemorySpace.HBM),
            pl.BlockSpec(memory_space=pltpu.MemorySpace.HBM),
        ],
        out_specs=pl.BlockSpec(memory_space=pltpu.VMEM),
        scratch_shapes=[
            pltpu.VMEM((SKV, H_PER, DH), jnp.float32),
            pltpu.VMEM((SKV, H_PER, DH), jnp.float32),
            pltpu.VMEM((SQ, D_MODEL), jnp.float32),
            pltpu.VMEM((N_DEV, ROWS, D_MODEL), jnp.float32),
            pltpu.SemaphoreType.DMA((2,)),
            pltpu.SemaphoreType.DMA((N_DEV,)),
            pltpu.SemaphoreType.DMA((N_DEV,)),
            pltpu.SemaphoreType.DMA((2, N_DEV)),
        ],
        compiler_params=pltpu.CompilerParams(
            vmem_limit_bytes=100 * 1024 * 1024),
    )(x, Wq, Wo, K_ext, V_ext)


# device time: 92539 ns/iter; 1.1230x vs baseline; 1.1230x over previous
import jax
import jax.numpy as jnp
from jax import lax
from jax.experimental import pallas as pl
from jax.experimental.pallas import tpu as pltpu

N_DEV = 16
SQ = 256
D_MODEL = 1024
SKV = 4096
H_PER = 8
DH = 128
ROWS = SQ // N_DEV
SCALE = 0.08838834764831843


def _bf(x):
    return x.astype(jnp.bfloat16)


def _fused_body(x_ref, wq_ref, wo_ref, k_hbm, v_hbm, out_ref,
                k_slab, v_slab, kh_buf, vh_buf, part_ref, p1_buf,
                copy_sems, ext_sems, p1_sems, p2_sems, send_sems):
    me = lax.axis_index("i")
    head0 = me * H_PER

    kcp = pltpu.make_async_copy(
        k_hbm.at[0, :, pl.ds(head0, H_PER), :], k_slab, copy_sems.at[0])
    vcp = pltpu.make_async_copy(
        v_hbm.at[0, :, pl.ds(head0, H_PER), :], v_slab, copy_sems.at[1])
    kcp.start()
    vcp.start()

    def extract(h, slot):
        ke = pltpu.make_async_copy(
            k_slab.at[:, h, :], kh_buf.at[slot], ext_sems.at[0, slot])
        ve = pltpu.make_async_copy(
            v_slab.at[:, h, :], vh_buf.at[slot], ext_sems.at[1, slot])
        return ke, ve

    q = jnp.dot(_bf(x_ref[0]), _bf(wq_ref[...]),
                preferred_element_type=jnp.float32)

    kcp.wait()
    vcp.wait()
    ke, ve = extract(0, 0)
    ke.start()
    ve.start()

    partial = jnp.zeros((SQ, D_MODEL), jnp.float32)
    for h in range(H_PER):
        slot = h % 2
        ke, ve = extract(h, slot)
        ke.wait()
        ve.wait()
        if h + 1 < H_PER:
            nke, nve = extract(h + 1, (h + 1) % 2)
            nke.start()
            nve.start()
        qh = _bf(q[:, h * DH:(h + 1) * DH])
        kh = _bf(kh_buf[slot])
        vh = _bf(vh_buf[slot])
        s = lax.dot_general(
            qh, kh, (((1,), (1,)), ((), ())),
            preferred_element_type=jnp.float32) * SCALE
        m = jnp.max(s, axis=1, keepdims=True)
        p = jnp.exp(s - m)
        l = jnp.sum(p, axis=1, keepdims=True)
        oh = jnp.dot(_bf(p), vh, preferred_element_type=jnp.float32) / l
        partial = partial + jnp.dot(
            _bf(oh), _bf(wo_ref[pl.ds(h * DH, DH), :]),
            preferred_element_type=jnp.float32)
    part_ref[...] = partial

    p1_descs = []
    for k in range(1, N_DEV):
        peer = lax.rem(me + k, N_DEV)
        d = pltpu.make_async_remote_copy(
            src_ref=part_ref.at[pl.ds(peer * ROWS, ROWS), :],
            dst_ref=p1_buf.at[N_DEV - k],
            send_sem=send_sems.at[0, k],
            recv_sem=p1_sems.at[N_DEV - k],
            device_id=(peer,),
            device_id_type=pl.DeviceIdType.MESH,
        )
        d.start()
        p1_descs.append(d)

    acc = part_ref[pl.ds(me * ROWS, ROWS), :]
    for k in range(1, N_DEV):
        r = pltpu.make_async_remote_copy(
            src_ref=part_ref.at[pl.ds(0, ROWS), :],
            dst_ref=p1_buf.at[k],
            send_sem=send_sems.at[0, 0],
            recv_sem=p1_sems.at[k],
            device_id=(me,),
            device_id_type=pl.DeviceIdType.MESH,
        )
        r.wait_recv()
        acc = acc + p1_buf[k]
    out_ref[0, pl.ds(me * ROWS, ROWS), :] = acc

    p2_descs = []
    for k in range(1, N_DEV):
        peer = lax.rem(me + k, N_DEV)
        d = pltpu.make_async_remote_copy(
            src_ref=out_ref.at[0, pl.ds(me * ROWS, ROWS), :],
            dst_ref=out_ref.at[0, pl.ds(me * ROWS, ROWS), :],
            send_sem=send_sems.at[1, k],
            recv_sem=p2_sems.at[N_DEV - k],
            device_id=(peer,),
            device_id_type=pl.DeviceIdType.MESH,
        )
        d.start()
        p2_descs.append(d)

    for k in range(1, N_DEV):
        src_chunk = lax.rem(me + k, N_DEV)
        r = pltpu.make_async_remote_copy(
            src_ref=out_ref.at[0, pl.ds(0, ROWS), :],
            dst_ref=out_ref.at[0, pl.ds(src_chunk * ROWS, ROWS), :],
            send_sem=send_sems.at[1, 0],
            recv_sem=p2_sems.at[k],
            device_id=(me,),
            device_id_type=pl.DeviceIdType.MESH,
        )
        r.wait_recv()

    for d in p1_descs + p2_descs:
        d.wait_send()


def kernel(x, Wq, Wo, K_ext, V_ext):
    return pl.pallas_call(
        _fused_body,
        out_shape=jax.ShapeDtypeStruct((1, SQ, D_MODEL), jnp.float32),
        in_specs=[
            pl.BlockSpec(memory_space=pltpu.VMEM),
            pl.BlockSpec(memory_space=pltpu.VMEM),
            pl.BlockSpec(memory_space=pltpu.VMEM),
            pl.BlockSpec(memory_space=pltpu.MemorySpace.HBM),
            pl.BlockSpec(memory_space=pltpu.MemorySpace.HBM),
        ],
        out_specs=pl.BlockSpec(memory_space=pltpu.VMEM),
        scratch_shapes=[
            pltpu.VMEM((SKV, H_PER, DH), jnp.float32),
            pltpu.VMEM((SKV, H_PER, DH), jnp.float32),
            pltpu.VMEM((2, SKV, DH), jnp.float32),
            pltpu.VMEM((2, SKV, DH), jnp.float32),
            pltpu.VMEM((SQ, D_MODEL), jnp.float32),
            pltpu.VMEM((N_DEV, ROWS, D_MODEL), jnp.float32),
            pltpu.SemaphoreType.DMA((2,)),
            pltpu.SemaphoreType.DMA((2, 2)),
            pltpu.SemaphoreType.DMA((N_DEV,)),
            pltpu.SemaphoreType.DMA((N_DEV,)),
            pltpu.SemaphoreType.DMA((2, N_DEV)),
        ],
        compiler_params=pltpu.CompilerParams(
            vmem_limit_bytes=100 * 1024 * 1024),
    )(x, Wq, Wo, K_ext, V_ext)


# device time: 52521 ns/iter; 1.9786x vs baseline; 1.7619x over previous
import jax
import jax.numpy as jnp
from jax import lax
from jax.experimental import pallas as pl
from jax.experimental.pallas import tpu as pltpu

N_DEV = 16
SQ = 256
D_MODEL = 1024
SKV = 4096
H_PER = 8
DH = 128
ROWS = SQ // N_DEV
SCALE = 0.08838834764831843


def _bf(x):
    return x.astype(jnp.bfloat16)


def _fused_body(x_ref, wq_ref, wo_ref, k_hbm, v_hbm, out_ref,
                k_bufs, v_bufs, part_bf, red_bf, p1_buf,
                copy_sems, p1_sems, p2_sems, send_sems):
    me = lax.axis_index("i")
    head0 = me * H_PER

    kcps, vcps = [], []
    for h in range(H_PER):
        kcp = pltpu.make_async_copy(
            k_hbm.at[0, :, head0 + h, :], k_bufs.at[h], copy_sems.at[0, h])
        vcp = pltpu.make_async_copy(
            v_hbm.at[0, :, head0 + h, :], v_bufs.at[h], copy_sems.at[1, h])
        kcp.start()
        vcp.start()
        kcps.append(kcp)
        vcps.append(vcp)

    q = jnp.dot(_bf(x_ref[0]), _bf(wq_ref[...]),
                preferred_element_type=jnp.float32)

    partial = jnp.zeros((SQ, D_MODEL), jnp.float32)
    for h in range(H_PER):
        kcps[h].wait()
        vcps[h].wait()
        qh = _bf(q[:, h * DH:(h + 1) * DH])
        kh = _bf(k_bufs[h])
        vh = _bf(v_bufs[h])
        s = lax.dot_general(
            qh, kh, (((1,), (1,)), ((), ())),
            preferred_element_type=jnp.float32) * SCALE
        m = jnp.max(s, axis=1, keepdims=True)
        p = jnp.exp(s - m)
        l = jnp.sum(p, axis=1, keepdims=True)
        oh = jnp.dot(_bf(p), vh, preferred_element_type=jnp.float32) / l
        partial = partial + jnp.dot(
            _bf(oh), _bf(wo_ref[pl.ds(h * DH, DH), :]),
            preferred_element_type=jnp.float32)
    part_bf[...] = _bf(partial)

    p1_descs = []
    for k in range(1, N_DEV):
        peer = lax.rem(me + k, N_DEV)
        d = pltpu.make_async_remote_copy(
            src_ref=part_bf.at[pl.ds(peer * ROWS, ROWS), :],
            dst_ref=p1_buf.at[N_DEV - k],
            send_sem=send_sems.at[0, k],
            recv_sem=p1_sems.at[N_DEV - k],
            device_id=(peer,),
            device_id_type=pl.DeviceIdType.MESH,
        )
        d.start()
        p1_descs.append(d)

    acc = part_bf[pl.ds(me * ROWS, ROWS), :].astype(jnp.float32)
    for k in range(1, N_DEV):
        r = pltpu.make_async_remote_copy(
            src_ref=part_bf.at[pl.ds(0, ROWS), :],
            dst_ref=p1_buf.at[k],
            send_sem=send_sems.at[0, 0],
            recv_sem=p1_sems.at[k],
            device_id=(me,),
            device_id_type=pl.DeviceIdType.MESH,
        )
        r.wait_recv()
        acc = acc + p1_buf[k].astype(jnp.float32)
    red_bf[pl.ds(me * ROWS, ROWS), :] = _bf(acc)

    p2_descs = []
    for k in range(1, N_DEV):
        peer = lax.rem(me + k, N_DEV)
        d = pltpu.make_async_remote_copy(
            src_ref=red_bf.at[pl.ds(me * ROWS, ROWS), :],
            dst_ref=red_bf.at[pl.ds(me * ROWS, ROWS), :],
            send_sem=send_sems.at[1, k],
            recv_sem=p2_sems.at[N_DEV - k],
            device_id=(peer,),
            device_id_type=pl.DeviceIdType.MESH,
        )
        d.start()
        p2_descs.append(d)

    for k in range(1, N_DEV):
        src_chunk = lax.rem(me + k, N_DEV)
        r = pltpu.make_async_remote_copy(
            src_ref=red_bf.at[pl.ds(0, ROWS), :],
            dst_ref=red_bf.at[pl.ds(src_chunk * ROWS, ROWS), :],
            send_sem=send_sems.at[1, 0],
            recv_sem=p2_sems.at[k],
            device_id=(me,),
            device_id_type=pl.DeviceIdType.MESH,
        )
        r.wait_recv()

    out_ref[0] = red_bf[...].astype(jnp.float32)

    for d in p1_descs + p2_descs:
        d.wait_send()


def kernel(x, Wq, Wo, K_ext, V_ext):
    return pl.pallas_call(
        _fused_body,
        out_shape=jax.ShapeDtypeStruct((1, SQ, D_MODEL), jnp.float32),
        in_specs=[
            pl.BlockSpec(memory_space=pltpu.VMEM),
            pl.BlockSpec(memory_space=pltpu.VMEM),
            pl.BlockSpec(memory_space=pltpu.VMEM),
            pl.BlockSpec(memory_space=pltpu.MemorySpace.HBM),
            pl.BlockSpec(memory_space=pltpu.MemorySpace.HBM),
        ],
        out_specs=pl.BlockSpec(memory_space=pltpu.VMEM),
        scratch_shapes=[
            pltpu.VMEM((H_PER, SKV, DH), jnp.float32),
            pltpu.VMEM((H_PER, SKV, DH), jnp.float32),
            pltpu.VMEM((SQ, D_MODEL), jnp.bfloat16),
            pltpu.VMEM((SQ, D_MODEL), jnp.bfloat16),
            pltpu.VMEM((N_DEV, ROWS, D_MODEL), jnp.bfloat16),
            pltpu.SemaphoreType.DMA((2, H_PER)),
            pltpu.SemaphoreType.DMA((N_DEV,)),
            pltpu.SemaphoreType.DMA((N_DEV,)),
            pltpu.SemaphoreType.DMA((2, N_DEV)),
        ],
        compiler_params=pltpu.CompilerParams(
            vmem_limit_bytes=100 * 1024 * 1024),
    )(x, Wq, Wo, K_ext, V_ext)
